# initial kernel scaffold (unmeasured)
import jax
import jax.numpy as jnp
from jax import lax
from jax.experimental import pallas as pl
from jax.experimental.pallas import tpu as pltpu

N_DEV = 4


def kernel(x, assign, W1, W2):
    T, D = x.shape
    E, _, F = W1.shape

    xb = x.astype(jnp.bfloat16)
    w1b = W1.astype(jnp.bfloat16)
    w2b = W2.astype(jnp.bfloat16)

    def body(x_ref, a_ref, w1_ref, w2_ref, out_ref,
             comm_x, comm_a, acc_own, acc_comm,
             xs_send, xs_recv, as_send, as_recv, acc_send, acc_recv):
        my = lax.axis_index("i")
        right = lax.rem(my + 1, N_DEV)
        left = lax.rem(my + N_DEV - 1, N_DEV)
        e0 = my * 2

        def ffw(xblk, ablk):
            out = None
            for e in range(E):
                m = ablk[:, None] == (e0 + e)
                xm = jnp.where(m, xblk, jnp.zeros_like(xblk))
                h = jnp.maximum(
                    jnp.dot(xm, w1_ref[e], preferred_element_type=jnp.float32),
                    0.0,
                ).astype(jnp.bfloat16)
                o = jnp.dot(h, w2_ref[e], preferred_element_type=jnp.float32)
                out = o if out is None else out + o
            return out

        def send_x(src, sslot, dslot):
            return pltpu.make_async_remote_copy(
                src_ref=src, dst_ref=comm_x.at[dslot],
                send_sem=xs_send.at[sslot], recv_sem=xs_recv.at[dslot],
                device_id=(right,), device_id_type=pl.DeviceIdType.MESH,
            )

        def send_a(src, sslot, dslot):
            return pltpu.make_async_remote_copy(
                src_ref=src, dst_ref=comm_a.at[dslot],
                send_sem=as_send.at[sslot], recv_sem=as_recv.at[dslot],
                device_id=(right,), device_id_type=pl.DeviceIdType.MESH,
            )

        def send_acc(src, sslot, dslot):
            return pltpu.make_async_remote_copy(
                src_ref=src, dst_ref=acc_comm.at[dslot],
                send_sem=acc_send.at[sslot], recv_sem=acc_recv.at[dslot],
                device_id=(right,), device_id_type=pl.DeviceIdType.MESH,
            )

        barrier = pltpu.get_barrier_semaphore()
        for nbr in (left, right):
            pl.semaphore_signal(barrier, inc=1, device_id=(nbr,),
                                device_id_type=pl.DeviceIdType.MESH)
        pl.semaphore_wait(barrier, 2)

        s_x0 = send_x(x_ref, 0, 0)
        s_a0 = send_a(a_ref, 0, 0)
        s_x0.start()
        s_a0.start()
        out_ref[...] = ffw(x_ref[...], a_ref[...])

        s_x0.wait_recv()
        s_a0.wait_recv()
        s_x1 = send_x(comm_x.at[0], 1, 1)
        s_a1 = send_a(comm_a.at[0], 1, 1)
        s_x1.start()
        s_a1.start()
        acc_own[...] = ffw(comm_x[0], comm_a[0]).astype(jnp.bfloat16)
        s_acc0 = send_acc(acc_own, 0, 0)
        s_acc0.start()

        s_x1.wait_recv()
        s_a1.wait_recv()
        s_x2 = send_x(comm_x.at[1], 2, 2)
        s_a2 = send_a(comm_a.at[1], 2, 2)
        s_x2.start()
        s_a2.start()
        a2 = ffw(comm_x[1], comm_a[1])
        s_acc0.wait_recv()
        acc_comm[0] = (a2 + acc_comm[0].astype(jnp.float32)).astype(jnp.bfloat16)
        s_acc1 = send_acc(acc_comm.at[0], 1, 1)
        s_acc1.start()

        s_x2.wait_recv()
        s_a2.wait_recv()
        a3 = ffw(comm_x[2], comm_a[2])
        s_acc1.wait_recv()
        acc_comm[1] = (a3 + acc_comm[1].astype(jnp.float32)).astype(jnp.bfloat16)
        s_acc2 = send_acc(acc_comm.at[1], 2, 2)
        s_acc2.start()

        s_acc2.wait_recv()
        out_ref[...] = out_ref[...] + acc_comm[2].astype(jnp.float32)

        for s in (s_x0, s_a0, s_x1, s_a1, s_x2, s_a2, s_acc0, s_acc1, s_acc2):
            s.wait_send()

    return pl.pallas_call(
        body,
        out_shape=jax.ShapeDtypeStruct((T, D), jnp.float32),
        in_specs=[
            pl.BlockSpec(memory_space=pltpu.VMEM),
            pl.BlockSpec(memory_space=pltpu.VMEM),
            pl.BlockSpec(memory_space=pltpu.VMEM),
            pl.BlockSpec(memory_space=pltpu.VMEM),
        ],
        out_specs=pl.BlockSpec(memory_space=pltpu.VMEM),
        scratch_shapes=[
            pltpu.VMEM((3, T, D), jnp.bfloat16),
            pltpu.VMEM((3, T), jnp.int32),
            pltpu.VMEM((T, D), jnp.bfloat16),
            pltpu.VMEM((3, T, D), jnp.bfloat16),
            pltpu.SemaphoreType.DMA((3,)),
            pltpu.SemaphoreType.DMA((3,)),
            pltpu.SemaphoreType.DMA((3,)),
            pltpu.SemaphoreType.DMA((3,)),
            pltpu.SemaphoreType.DMA((3,)),
            pltpu.SemaphoreType.DMA((3,)),
        ],
        compiler_params=pltpu.CompilerParams(collective_id=0),
    )(xb, assign, w1b, w2b)


# baseline (device time: 184163 ns/iter reference)
import jax
import jax.numpy as jnp
from jax import lax
from jax.experimental import pallas as pl
from jax.experimental.pallas import tpu as pltpu

N_DEV = 4


def kernel(x, assign, W1, W2):
    T, D = x.shape
    E, _, F = W1.shape

    xb = x.astype(jnp.bfloat16)
    w1b = W1.astype(jnp.bfloat16)
    w2b = W2.astype(jnp.bfloat16)
    a2d = assign.reshape(T, 1)

    def body(x_ref, a_ref, w1_ref, w2_ref, out_ref,
             comm_x, comm_a, acc_own, acc_comm,
             xs_send, xs_recv, as_send, as_recv, acc_send, acc_recv):
        my = lax.axis_index("i")
        right = lax.rem(my + 1, N_DEV)
        left = lax.rem(my + N_DEV - 1, N_DEV)
        e0 = my * 2

        FC = F // 2

        def ffw(xblk, ablk):
            out = None
            for e in range(E):
                m = ablk == (e0 + e)
                xm = jnp.where(m, xblk, jnp.zeros_like(xblk))
                for c in range(F // FC):
                    h = jnp.maximum(
                        jnp.dot(
                            xm,
                            w1_ref[e, :, c * FC:(c + 1) * FC],
                            preferred_element_type=jnp.float32,
                        ),
                        0.0,
                    ).astype(jnp.bfloat16)
                    o = jnp.dot(
                        h,
                        w2_ref[e, c * FC:(c + 1) * FC, :],
                        preferred_element_type=jnp.float32,
                    )
                    out = o if out is None else out + o
            return out

        def send_x(src, sslot, dslot):
            return pltpu.make_async_remote_copy(
                src_ref=src, dst_ref=comm_x.at[dslot],
                send_sem=xs_send.at[sslot], recv_sem=xs_recv.at[dslot],
                device_id=(right,), device_id_type=pl.DeviceIdType.MESH,
            )

        def send_a(src, sslot, dslot):
            return pltpu.make_async_remote_copy(
                src_ref=src, dst_ref=comm_a.at[dslot],
                send_sem=as_send.at[sslot], recv_sem=as_recv.at[dslot],
                device_id=(right,), device_id_type=pl.DeviceIdType.MESH,
            )

        def send_acc(src, sslot, dslot):
            return pltpu.make_async_remote_copy(
                src_ref=src, dst_ref=acc_comm.at[dslot],
                send_sem=acc_send.at[sslot], recv_sem=acc_recv.at[dslot],
                device_id=(right,), device_id_type=pl.DeviceIdType.MESH,
            )

        barrier = pltpu.get_barrier_semaphore()
        for nbr in (left, right):
            pl.semaphore_signal(barrier, inc=1, device_id=(nbr,),
                                device_id_type=pl.DeviceIdType.MESH)
        pl.semaphore_wait(barrier, 2)

        s_x0 = send_x(x_ref, 0, 0)
        s_a0 = send_a(a_ref, 0, 0)
        s_x0.start()
        s_a0.start()
        out_ref[...] = ffw(x_ref[...], a_ref[...])

        s_x0.wait_recv()
        s_a0.wait_recv()
        s_x1 = send_x(comm_x.at[0], 1, 1)
        s_a1 = send_a(comm_a.at[0], 1, 1)
        s_x1.start()
        s_a1.start()
        acc_own[...] = ffw(comm_x[0], comm_a[0]).astype(jnp.bfloat16)
        s_acc0 = send_acc(acc_own, 0, 0)
        s_acc0.start()

        s_x1.wait_recv()
        s_a1.wait_recv()
        s_x2 = send_x(comm_x.at[1], 2, 2)
        s_a2 = send_a(comm_a.at[1], 2, 2)
        s_x2.start()
        s_a2.start()
        a2 = ffw(comm_x[1], comm_a[1])
        s_acc0.wait_recv()
        acc_comm[0] = (a2 + acc_comm[0].astype(jnp.float32)).astype(jnp.bfloat16)
        s_acc1 = send_acc(acc_comm.at[0], 1, 1)
        s_acc1.start()

        s_x2.wait_recv()
        s_a2.wait_recv()
        a3 = ffw(comm_x[2], comm_a[2])
        s_acc1.wait_recv()
        acc_comm[1] = (a3 + acc_comm[1].astype(jnp.float32)).astype(jnp.bfloat16)
        s_acc2 = send_acc(acc_comm.at[1], 2, 2)
        s_acc2.start()

        s_acc2.wait_recv()
        out_ref[...] = out_ref[...] + acc_comm[2].astype(jnp.float32)

        for s in (s_x0, s_a0, s_x1, s_a1, s_x2, s_a2, s_acc0, s_acc1, s_acc2):
            s.wait_send()

    return pl.pallas_call(
        body,
        out_shape=jax.ShapeDtypeStruct((T, D), jnp.float32),
        in_specs=[
            pl.BlockSpec(memory_space=pltpu.VMEM),
            pl.BlockSpec(memory_space=pltpu.VMEM),
            pl.BlockSpec(memory_space=pltpu.VMEM),
            pl.BlockSpec(memory_space=pltpu.VMEM),
        ],
        out_specs=pl.BlockSpec(memory_space=pltpu.VMEM),
        scratch_shapes=[
            pltpu.VMEM((3, T, D), jnp.bfloat16),
            pltpu.VMEM((3, T, 1), jnp.int32),
            pltpu.VMEM((T, D), jnp.bfloat16),
            pltpu.VMEM((3, T, D), jnp.bfloat16),
            pltpu.SemaphoreType.DMA((3,)),
            pltpu.SemaphoreType.DMA((3,)),
            pltpu.SemaphoreType.DMA((3,)),
            pltpu.SemaphoreType.DMA((3,)),
            pltpu.SemaphoreType.DMA((3,)),
            pltpu.SemaphoreType.DMA((3,)),
        ],
        compiler_params=pltpu.CompilerParams(collective_id=0),
    )(xb, a2d, w1b, w2b)


# device time: 119566 ns/iter; 1.5403x vs baseline; 1.5403x over previous
import jax
import jax.numpy as jnp
from jax import lax
from jax.experimental import pallas as pl
from jax.experimental.pallas import tpu as pltpu

N_DEV = 4


def kernel(x, assign, W1, W2):
    T, D = x.shape
    E, _, F = W1.shape
    T2 = T // 2

    xb = x.astype(jnp.bfloat16)
    w1b = W1.astype(jnp.bfloat16)
    w2b = W2.astype(jnp.bfloat16)
    a2d = assign.reshape(T, 1)

    def body(x_ref, a_ref, w1_ref, w2_ref, out_ref,
             comm_lo, comm_hi, ca_lo, ca_hi,
             accown_lo, accown_hi, acc_lo, acc_hi,
             xlo_s, xlo_r, xhi_s, xhi_r,
             alo_s, alo_r, ahi_s, ahi_r,
             clo_s, clo_r, chi_s, chi_r):
        my = lax.axis_index("i")
        right = lax.rem(my + 1, N_DEV)
        left = lax.rem(my + N_DEV - 1, N_DEV)
        e0 = my * 2

        FC = F // 2

        def ffw(xblk, ablk):
            out = None
            for e in range(E):
                m = ablk == (e0 + e)
                xm = jnp.where(m, xblk, jnp.zeros_like(xblk))
                for c in range(F // FC):
                    h = jnp.maximum(
                        jnp.dot(
                            xm,
                            w1_ref[e, :, c * FC:(c + 1) * FC],
                            preferred_element_type=jnp.float32,
                        ),
                        0.0,
                    ).astype(jnp.bfloat16)
                    o = jnp.dot(
                        h,
                        w2_ref[e, c * FC:(c + 1) * FC, :],
                        preferred_element_type=jnp.float32,
                    )
                    out = o if out is None else out + o
            return out

        def rc(src, dst, ssem, rsem, tgt):
            return pltpu.make_async_remote_copy(
                src_ref=src, dst_ref=dst, send_sem=ssem, recv_sem=rsem,
                device_id=(tgt,), device_id_type=pl.DeviceIdType.MESH,
            )

        barrier = pltpu.get_barrier_semaphore()
        for nbr in (left, right):
            pl.semaphore_signal(barrier, inc=1, device_id=(nbr,),
                                device_id_type=pl.DeviceIdType.MESH)
        pl.semaphore_wait(barrier, 2)

        s_xlo0 = rc(x_ref.at[pl.ds(0, T2), :], comm_lo.at[0],
                    xlo_s.at[0], xlo_r.at[0], right)
        s_xhi0 = rc(x_ref.at[pl.ds(T2, T2), :], comm_hi.at[0],
                    xhi_s.at[0], xhi_r.at[0], left)
        s_alo0 = rc(a_ref.at[pl.ds(0, T2), :], ca_lo.at[0],
                    alo_s.at[0], alo_r.at[0], right)
        s_ahi0 = rc(a_ref.at[pl.ds(T2, T2), :], ca_hi.at[0],
                    ahi_s.at[0], ahi_r.at[0], left)
        for s in (s_xlo0, s_xhi0, s_alo0, s_ahi0):
            s.start()
        out_ref[...] = ffw(x_ref[...], a_ref[...])

        s_xlo0.wait_recv()
        s_alo0.wait_recv()
        s_xlo1 = rc(comm_lo.at[0], comm_lo.at[1], xlo_s.at[1], xlo_r.at[1], right)
        s_alo1 = rc(ca_lo.at[0], ca_lo.at[1], alo_s.at[1], alo_r.at[1], right)
        s_xlo1.start()
        s_alo1.start()
        s_xhi0.wait_recv()
        s_ahi0.wait_recv()
        s_xhi1 = rc(comm_hi.at[0], comm_hi.at[1], xhi_s.at[1], xhi_r.at[1], left)
        s_ahi1 = rc(ca_hi.at[0], ca_hi.at[1], ahi_s.at[1], ahi_r.at[1], left)
        s_xhi1.start()
        s_ahi1.start()
        accown_lo[...] = ffw(comm_lo[0], ca_lo[0]).astype(jnp.bfloat16)
        s_clo0 = rc(accown_lo, acc_lo.at[0], clo_s.at[0], clo_r.at[0], right)
        s_clo0.start()
        accown_hi[...] = ffw(comm_hi[0], ca_hi[0]).astype(jnp.bfloat16)
        s_chi0 = rc(accown_hi, acc_hi.at[0], chi_s.at[0], chi_r.at[0], left)
        s_chi0.start()

        s_xlo1.wait_recv()
        s_alo1.wait_recv()
        s_xlo2 = rc(comm_lo.at[1], comm_lo.at[2], xlo_s.at[2], xlo_r.at[2], right)
        s_alo2 = rc(ca_lo.at[1], ca_lo.at[2], alo_s.at[2], alo_r.at[2], right)
        s_xlo2.start()
        s_alo2.start()
        s_xhi1.wait_recv()
        s_ahi1.wait_recv()
        s_xhi2 = rc(comm_hi.at[1], comm_hi.at[2], xhi_s.at[2], xhi_r.at[2], left)
        s_ahi2 = rc(ca_hi.at[1], ca_hi.at[2], ahi_s.at[2], ahi_r.at[2], left)
        s_xhi2.start()
        s_ahi2.start()
        a2lo = ffw(comm_lo[1], ca_lo[1])
        s_clo0.wait_recv()
        acc_lo[0] = (a2lo + acc_lo[0].astype(jnp.float32)).astype(jnp.bfloat16)
        s_clo1 = rc(acc_lo.at[0], acc_lo.at[1], clo_s.at[1], clo_r.at[1], right)
        s_clo1.start()
        a2hi = ffw(comm_hi[1], ca_hi[1])
        s_chi0.wait_recv()
        acc_hi[0] = (a2hi + acc_hi[0].astype(jnp.float32)).astype(jnp.bfloat16)
        s_chi1 = rc(acc_hi.at[0], acc_hi.at[1], chi_s.at[1], chi_r.at[1], left)
        s_chi1.start()

        s_xlo2.wait_recv()
        s_alo2.wait_recv()
        a3lo = ffw(comm_lo[2], ca_lo[2])
        s_clo1.wait_recv()
        acc_lo[1] = (a3lo + acc_lo[1].astype(jnp.float32)).astype(jnp.bfloat16)
        s_clo2 = rc(acc_lo.at[1], acc_lo.at[2], clo_s.at[2], clo_r.at[2], right)
        s_clo2.start()
        s_xhi2.wait_recv()
        s_ahi2.wait_recv()
        a3hi = ffw(comm_hi[2], ca_hi[2])
        s_chi1.wait_recv()
        acc_hi[1] = (a3hi + acc_hi[1].astype(jnp.float32)).astype(jnp.bfloat16)
        s_chi2 = rc(acc_hi.at[1], acc_hi.at[2], chi_s.at[2], chi_r.at[2], left)
        s_chi2.start()

        s_clo2.wait_recv()
        out_ref[pl.ds(0, T2), :] = (
            out_ref[pl.ds(0, T2), :] + acc_lo[2].astype(jnp.float32)
        )
        s_chi2.wait_recv()
        out_ref[pl.ds(T2, T2), :] = (
            out_ref[pl.ds(T2, T2), :] + acc_hi[2].astype(jnp.float32)
        )

        for s in (s_xlo0, s_xhi0, s_alo0, s_ahi0,
                  s_xlo1, s_xhi1, s_alo1, s_ahi1,
                  s_xlo2, s_xhi2, s_alo2, s_ahi2,
                  s_clo0, s_chi0, s_clo1, s_chi1, s_clo2, s_chi2):
            s.wait_send()

    return pl.pallas_call(
        body,
        out_shape=jax.ShapeDtypeStruct((T, D), jnp.float32),
        in_specs=[
            pl.BlockSpec(memory_space=pltpu.VMEM),
            pl.BlockSpec(memory_space=pltpu.VMEM),
            pl.BlockSpec(memory_space=pltpu.VMEM),
            pl.BlockSpec(memory_space=pltpu.VMEM),
        ],
        out_specs=pl.BlockSpec(memory_space=pltpu.VMEM),
        scratch_shapes=[
            pltpu.VMEM((3, T2, D), jnp.bfloat16),
            pltpu.VMEM((3, T2, D), jnp.bfloat16),
            pltpu.VMEM((3, T2, 1), jnp.int32),
            pltpu.VMEM((3, T2, 1), jnp.int32),
            pltpu.VMEM((T2, D), jnp.bfloat16),
            pltpu.VMEM((T2, D), jnp.bfloat16),
            pltpu.VMEM((3, T2, D), jnp.bfloat16),
            pltpu.VMEM((3, T2, D), jnp.bfloat16),
            pltpu.SemaphoreType.DMA((3,)),
            pltpu.SemaphoreType.DMA((3,)),
            pltpu.SemaphoreType.DMA((3,)),
            pltpu.SemaphoreType.DMA((3,)),
            pltpu.SemaphoreType.DMA((3,)),
            pltpu.SemaphoreType.DMA((3,)),
            pltpu.SemaphoreType.DMA((3,)),
            pltpu.SemaphoreType.DMA((3,)),
            pltpu.SemaphoreType.DMA((3,)),
            pltpu.SemaphoreType.DMA((3,)),
            pltpu.SemaphoreType.DMA((3,)),
            pltpu.SemaphoreType.DMA((3,)),
        ],
        compiler_params=pltpu.CompilerParams(collective_id=0),
    )(xb, a2d, w1b, w2b)
